# baseline (speedup 1.0000x reference)
import functools

import jax
import jax.numpy as jnp
from jax import lax
from jax.experimental import pallas as pl
from jax.experimental.pallas import tpu as pltpu
from jax.experimental.pallas import tpu_sc as plsc


def _gather_kernel(V, B0, K, cols_per_w, NC, NS):
    mesh = plsc.VectorSubcoreMesh(core_axis_name="c", subcore_axis_name="s")
    PS = 10000
    assert V % PS == 0 and PS % 8 == 0
    NP = V // PS
    max_i = (NP + NS - 1) // NS
    CW = 128
    assert cols_per_w % CW == 0
    n_chunks = cols_per_w // CW

    @functools.partial(
        pl.kernel,
        mesh=mesh,
        out_type=jax.ShapeDtypeStruct((K, B0), jnp.float32),
        scratch_types=[
            pltpu.VMEM_SHARED((V,), jnp.float32),
            pltpu.VMEM((PS,), jnp.float32),
            pltpu.VMEM((K, CW), jnp.int32),
            pltpu.VMEM((K, CW), jnp.float32),
            pltpu.SemaphoreType.DMA,
        ],
        compiler_params=pltpu.CompilerParams(use_tc_tiling_on_sc=True),
    )
    def k(table_hbm, idx_hbm, out_hbm, shared, stage_v, idx_v, vals_v, sem):
        c = lax.axis_index("c")
        s = lax.axis_index("s")
        wid = s * NC + c
        for i in range(max_i):
            p = i * NS + s

            @pl.when(p < NP)
            def _():
                off = p * PS
                pltpu.sync_copy(table_hbm.at[pl.ds(off, PS)], stage_v)
                pltpu.sync_copy(stage_v, shared.at[pl.ds(off, PS)])

        plsc.subcore_barrier()
        for ch in range(n_chunks):
            col0 = wid * cols_per_w + ch * CW
            pltpu.sync_copy(idx_hbm.at[:, pl.ds(col0, CW)], idx_v)
            copies = [
                pltpu.async_copy(shared.at[idx_v.at[j]], vals_v.at[j], sem)
                for j in range(K)
            ]
            for cp in copies:
                cp.wait()
            pltpu.sync_copy(vals_v, out_hbm.at[:, pl.ds(col0, CW)])

    return k


def kernel(fit_X_col, donors_idx):
    B0, K = donors_idx.shape
    V = fit_X_col.shape[0]
    info = plsc.get_sparse_core_info()
    NC, NS = info.num_cores, info.num_subcores
    NW = NC * NS
    assert B0 % (8 * NW) == 0
    cols_per_w = B0 // NW
    idx_t = donors_idx.astype(jnp.int32).T
    out_t = _gather_kernel(V, B0, K, cols_per_w, NC, NS)(fit_X_col, idx_t)
    return out_t.T


# speedup vs baseline: 1.2442x; 1.2442x over previous
import functools

import jax
import jax.numpy as jnp
from jax import lax
from jax.experimental import pallas as pl
from jax.experimental.pallas import tpu as pltpu
from jax.experimental.pallas import tpu_sc as plsc


def _gather_kernel(V, B0, K, NC, NS):
    mesh = plsc.VectorSubcoreMesh(core_axis_name="c", subcore_axis_name="s")
    NW = NC * NS
    P = 2048
    PPR = B0 // P
    NPIECE = K * PPR
    nfull = NPIECE // NW
    nrem = NPIECE % NW
    maxp = nfull + (1 if nrem else 0)
    PS = 10000
    assert V % PS == 0 and PS % 8 == 0 and PS <= nfull * P
    NP = V // PS
    max_i = (NP + NS - 1) // NS

    @functools.partial(
        pl.kernel,
        mesh=mesh,
        out_type=jax.ShapeDtypeStruct((K, B0), jnp.float32),
        scratch_types=[
            pltpu.VMEM_SHARED((V,), jnp.float32),
            pltpu.VMEM((maxp * P,), jnp.int32),
            pltpu.VMEM((maxp * P,), jnp.float32),
            pltpu.SemaphoreType.DMA,
            pltpu.SemaphoreType.DMA,
        ],
        compiler_params=pltpu.CompilerParams(use_tc_tiling_on_sc=True),
    )
    def k(table_hbm, idx_hbm, out_hbm, shared, idx_v, vals_v, sem, sem2):
        c = lax.axis_index("c")
        s = lax.axis_index("s")
        wid = s * NC + c

        idx_copies = []
        for i in range(nfull):
            q = i * NW + wid
            j = q // PPR
            col = (q % PPR) * P
            idx_copies.append(
                pltpu.async_copy(
                    idx_hbm.at[j, pl.ds(col, P)],
                    idx_v.at[pl.ds(i * P, P)],
                    sem2,
                )
            )

        for i in range(max_i):
            p = i * NS + s

            @pl.when(p < NP)
            def _():
                off = p * PS
                pltpu.sync_copy(table_hbm.at[pl.ds(off, PS)], vals_v.at[pl.ds(0, PS)])
                pltpu.sync_copy(vals_v.at[pl.ds(0, PS)], shared.at[pl.ds(off, PS)])

        @pl.when(wid < nrem)
        def _():
            q = nfull * NW + wid
            j = q // PPR
            col = (q % PPR) * P
            pltpu.sync_copy(
                idx_hbm.at[j, pl.ds(col, P)],
                idx_v.at[pl.ds(nfull * P, P)],
            )

        for cp in idx_copies:
            cp.wait()
        plsc.subcore_barrier()

        pltpu.async_copy(
            shared.at[idx_v.at[pl.ds(0, nfull * P)]],
            vals_v.at[pl.ds(0, nfull * P)],
            sem,
        ).wait()

        @pl.when(wid < nrem)
        def _():
            pltpu.async_copy(
                shared.at[idx_v.at[pl.ds(nfull * P, P)]],
                vals_v.at[pl.ds(nfull * P, P)],
                sem,
            ).wait()

        out_copies = []
        for i in range(nfull):
            q = i * NW + wid
            j = q // PPR
            col = (q % PPR) * P
            out_copies.append(
                pltpu.async_copy(
                    vals_v.at[pl.ds(i * P, P)],
                    out_hbm.at[j, pl.ds(col, P)],
                    sem2,
                )
            )

        @pl.when(wid < nrem)
        def _():
            q = nfull * NW + wid
            j = q // PPR
            col = (q % PPR) * P
            pltpu.sync_copy(
                vals_v.at[pl.ds(nfull * P, P)],
                out_hbm.at[j, pl.ds(col, P)],
            )

        for cp in out_copies:
            cp.wait()

    return k


def kernel(fit_X_col, donors_idx):
    B0, K = donors_idx.shape
    V = fit_X_col.shape[0]
    info = plsc.get_sparse_core_info()
    NC, NS = info.num_cores, info.num_subcores
    idx_t = donors_idx.astype(jnp.int32).T
    out_t = _gather_kernel(V, B0, K, NC, NS)(fit_X_col, idx_t)
    return out_t.T
